# baseline (speedup 1.0000x reference)
import jax
import jax.numpy as jnp
from jax import lax
from jax.experimental import pallas as pl
from jax.experimental.pallas import tpu as pltpu
from jax.experimental.pallas import tpu_sc as plsc

_NUM_LABELS = 81
_POS_K = 170
_NEG_K = 341
_TH = 0.5
_RNG_SEED = 42
_NPAD = 5008


def _phase1_body(roi_t_ref, gt_ref, glab_ref, mrg_out_ref, lab_out_ref, dlt_out_ref):
    r = roi_t_ref[0]
    n = r.shape[1]
    by1, bx1, by2, bx2 = r[0:1, :], r[1:2, :], r[2:3, :], r[3:4, :]
    g = gt_ref[0]
    m = g.shape[0]
    gy1, gx1, gy2, gx2 = g[:, 0:1], g[:, 1:2], g[:, 2:3], g[:, 3:4]
    glab = glab_ref[0]

    x_top = jnp.maximum(bx1, gx1)
    y_top = jnp.maximum(by1, gy1)
    x_bot = jnp.minimum(bx2, gx2)
    y_bot = jnp.minimum(by2, gy2)
    inter = jnp.maximum(x_bot - x_top, 0.0) * jnp.maximum(y_bot - y_top, 0.0)
    barea = (by2 - by1) * (bx2 - bx1)
    garea = (gy2 - gy1) * (gx2 - gx1)
    union = barea + garea - inter
    iou = inter / jnp.maximum(union, 1e-7)

    merged = jnp.max(iou, axis=0, keepdims=True)
    iota_g = lax.broadcasted_iota(jnp.int32, (m, n), 0)
    amax = jnp.min(jnp.where(iou == merged, iota_g, m), axis=0, keepdims=True)
    onehot = iota_g == amax

    egy1 = jnp.sum(jnp.where(onehot, gy1, 0.0), axis=0, keepdims=True)
    egx1 = jnp.sum(jnp.where(onehot, gx1, 0.0), axis=0, keepdims=True)
    egy2 = jnp.sum(jnp.where(onehot, gy2, 0.0), axis=0, keepdims=True)
    egx2 = jnp.sum(jnp.where(onehot, gx2, 0.0), axis=0, keepdims=True)
    elab = jnp.sum(jnp.where(onehot, glab, 0), axis=0, keepdims=True)

    mrg_out_ref[0] = merged
    lab_out_ref[0] = elab

    bw = bx2 - bx1
    bh = by2 - by1
    bcx = bx1 + 0.5 * bw
    bcy = by1 + 0.5 * bh
    gw = egx2 - egx1
    gh = egy2 - egy1
    gcx = egx1 + 0.5 * gw
    gcy = egy1 + 0.5 * gh
    bw = jnp.where(bw == 0.0, 0.001, bw)
    bh = jnp.where(bh == 0.0, 0.001, bh)
    dx = jnp.where(gw == 0.0, 0.0, (gcx - bcx) / bw) * 10.0
    dy = jnp.where(gh == 0.0, 0.0, (gcy - bcy) / bh) * 10.0
    dw = jnp.where(gw == 0.0, 0.0, jnp.log(jnp.where(gw == 0.0, 1.0, gw / bw))) * 5.0
    dh = jnp.where(gh == 0.0, 0.0, jnp.log(jnp.where(gh == 0.0, 1.0, gh / bh))) * 5.0
    dlt_out_ref[0] = jnp.concatenate([dy, dx, dh, dw], axis=0)


_GATHER_DNUMS = lax.GatherDimensionNumbers(
    offset_dims=(), collapsed_slice_dims=(0,), start_index_map=(0,))


def _perm(x, idx):
    return lax.gather(x, idx[:, None], _GATHER_DNUMS, (1,),
                      mode=lax.GatherScatterMode.PROMISE_IN_BOUNDS,
                      unique_indices=True, indices_are_sorted=False)


def _sc_select_body(mrg_hbm, rnd_hbm, sel_hbm, mrg_v, rnd_v, mult_v, sel_v):
    wid = lax.axis_index("s") * 2 + lax.axis_index("c")
    nv = _NPAD // 16
    iota16 = lax.iota(jnp.int32, 16)

    def splat_sum(x):
        for sft in (1, 2, 4, 8):
            x = x + _perm(x, iota16 ^ sft)
        return x

    def prefix_incl(x):
        for sft in (1, 2, 4, 8):
            shifted = _perm(x, jnp.maximum(iota16 - sft, 0))
            x = x + jnp.where(iota16 >= sft, shifted, 0)
        return x

    @pl.when(wid < 16)
    def _():
        b = wid >> 1
        kind = wid % 2
        pltpu.sync_copy(mrg_hbm.at[b], mrg_v)
        pltpu.sync_copy(rnd_hbm.at[kind, b], rnd_v)
        k = _POS_K * (1 - kind) + _NEG_K * kind
        kpi = 1 - kind

        def precompute(i, c):
            mrg = mrg_v[pl.ds(i * 16, 16)]
            rnd = rnd_v[pl.ds(i * 16, 16)]
            mp = jnp.where(mrg > _TH, 1, 0)
            mn = jnp.where(mrg < _TH, 1, 0) * jnp.where(mrg >= 0.0, 1, 0)
            msk_i = kpi * mp + (1 - kpi) * mn
            mult_v[pl.ds(i * 16, 16)] = rnd * msk_i
            return c

        lax.fori_loop(0, nv, precompute, jnp.int32(0))

        def count_ge(t):
            def body(i, acc):
                mult = mult_v[pl.ds(i * 16, 16)]
                return acc + jnp.where(mult >= t, 1, 0)
            acc = lax.fori_loop(0, nv, body, jnp.zeros((16,), jnp.int32))
            return splat_sum(acc)

        def bs_body(_, lohi):
            lo, hi = lohi
            mid = (lo + hi) >> 1
            take = count_ge(mid) >= k
            return jnp.where(take, mid, lo), jnp.where(take, hi, mid)

        lo0 = jnp.zeros((16,), jnp.int32)
        hi0 = jnp.zeros((16,), jnp.int32) + 4096
        t, _ = lax.fori_loop(0, 12, bs_body, (lo0, hi0))
        r = k - count_ge(t + 1)

        def tie_body(i, carry):
            mult = mult_v[pl.ds(i * 16, 16)]
            nz_i = jnp.where(mult > 0, 1, 0)
            eq_i = jnp.where(mult == t, 1, 0)
            pref = prefix_incl(eq_i)
            tie_i = eq_i * jnp.where((carry + pref) <= r, 1, 0)
            gt_i = jnp.where(mult > t, 1, 0)
            sel_v[pl.ds(i * 16, 16)] = nz_i * (gt_i + tie_i)
            return carry + splat_sum(eq_i)

        lax.fori_loop(0, nv, tie_body, jnp.zeros((16,), jnp.int32))
        pltpu.sync_copy(sel_v, sel_hbm.at[kind, b])


def _phase2_body(lab_ref, dlt_ref, selp_ref, seln_ref, oh_ref,
                 d0_ref, d1_ref, d2_ref, d3_ref):
    lab_g = lab_ref[0]
    dlt = dlt_ref[0]
    sp = selp_ref[0] != 0
    sn = seln_ref[0] != 0
    t = lab_g.shape[0]
    lab = jnp.where(sp, lab_g, -1) + sn.astype(jnp.int32)
    iota_c = lax.broadcasted_iota(jnp.int32, (t, _NUM_LABELS), 1)
    hit = iota_c == lab
    oh_ref[0] = hit.astype(jnp.int32)
    hitp = hit & sp
    d0_ref[0] = jnp.where(hitp, dlt[:, 0:1], 0.0)
    d1_ref[0] = jnp.where(hitp, dlt[:, 1:2], 0.0)
    d2_ref[0] = jnp.where(hitp, dlt[:, 2:3], 0.0)
    d3_ref[0] = jnp.where(hitp, dlt[:, 3:4], 0.0)


def kernel(roi_bboxes, gt_boxes, gt_labels):
    b, n = roi_bboxes.shape[0], roi_bboxes.shape[1]
    m = gt_boxes.shape[1]

    key42 = jax.random.key(_RNG_SEED)
    rand_p = jax.random.randint(key42, (b, n), 1, _POS_K * 10, dtype=jnp.int32)
    rand_n = jax.random.randint(key42, (b, n), 1, _NEG_K * 10, dtype=jnp.int32)
    rnd = jnp.stack([rand_p, rand_n])
    rnd = jnp.pad(rnd, ((0, 0), (0, 0), (0, _NPAD - n)))

    roi_t = jnp.transpose(roi_bboxes, (0, 2, 1))
    glab3 = gt_labels.reshape(b, m, 1)

    mrg_row, lab_row, dlt_row = pl.pallas_call(
        _phase1_body,
        grid=(b,),
        in_specs=[
            pl.BlockSpec((1, 4, n), lambda i: (i, 0, 0)),
            pl.BlockSpec((1, m, 4), lambda i: (i, 0, 0)),
            pl.BlockSpec((1, m, 1), lambda i: (i, 0, 0)),
        ],
        out_specs=[
            pl.BlockSpec((1, 1, n), lambda i: (i, 0, 0)),
            pl.BlockSpec((1, 1, n), lambda i: (i, 0, 0)),
            pl.BlockSpec((1, 4, n), lambda i: (i, 0, 0)),
        ],
        out_shape=[
            jax.ShapeDtypeStruct((b, 1, n), jnp.float32),
            jax.ShapeDtypeStruct((b, 1, n), jnp.int32),
            jax.ShapeDtypeStruct((b, 4, n), jnp.float32),
        ],
    )(roi_t, gt_boxes, glab3)

    mrg_pad = jnp.pad(mrg_row.reshape(b, n), ((0, 0), (0, _NPAD - n)),
                      constant_values=-1.0)

    mesh = plsc.VectorSubcoreMesh(core_axis_name="c", subcore_axis_name="s")
    sel = pl.kernel(
        _sc_select_body,
        mesh=mesh,
        out_type=jax.ShapeDtypeStruct((2, b, _NPAD), jnp.int32),
        scratch_types=[
            pltpu.VMEM((_NPAD,), jnp.float32),
            pltpu.VMEM((_NPAD,), jnp.int32),
            pltpu.VMEM((_NPAD,), jnp.int32),
            pltpu.VMEM((_NPAD,), jnp.int32),
        ],
    )(mrg_pad, rnd)

    selp = sel[0, :, :n].reshape(b, n, 1)
    seln = sel[1, :, :n].reshape(b, n, 1)
    lab_col = jnp.transpose(lab_row, (0, 2, 1))
    dlt_col = jnp.transpose(dlt_row, (0, 2, 1))

    tile = 1000
    lab_spec = pl.BlockSpec((1, tile, _NUM_LABELS), lambda i, j: (i, j, 0))
    oh, d0, d1, d2, d3 = pl.pallas_call(
        _phase2_body,
        grid=(b, n // tile),
        in_specs=[
            pl.BlockSpec((1, tile, 1), lambda i, j: (i, j, 0)),
            pl.BlockSpec((1, tile, 4), lambda i, j: (i, j, 0)),
            pl.BlockSpec((1, tile, 1), lambda i, j: (i, j, 0)),
            pl.BlockSpec((1, tile, 1), lambda i, j: (i, j, 0)),
        ],
        out_specs=[lab_spec] * 5,
        out_shape=[
            jax.ShapeDtypeStruct((b, n, _NUM_LABELS), jnp.int32),
        ] + [jax.ShapeDtypeStruct((b, n, _NUM_LABELS), jnp.float32)] * 4,
    )(lab_col, dlt_col, selp, seln)

    e = jnp.eye(4, dtype=jnp.float32)
    shape4 = (b, n, _NUM_LABELS, 4)
    big = sum(
        lax.broadcast_in_dim(dd, shape4, (0, 1, 2)) * e[i]
        for i, dd in enumerate((d0, d1, d2, d3)))
    return big.reshape(b, n * _NUM_LABELS, 4), oh


# speedup vs baseline: 1.1792x; 1.1792x over previous
import jax
import jax.numpy as jnp
from jax import lax
from jax.experimental import pallas as pl
from jax.experimental.pallas import tpu as pltpu
from jax.experimental.pallas import tpu_sc as plsc

_NUM_LABELS = 81
_POS_K = 170
_NEG_K = 341
_TH = 0.5
_RNG_SEED = 42
_NPAD = 5008


def _phase1_body(roi_t_ref, gt_ref, glab_ref, mrg_out_ref, lab_out_ref, dlt_out_ref):
    r = roi_t_ref[0]
    n = r.shape[1]
    by1, bx1, by2, bx2 = r[0:1, :], r[1:2, :], r[2:3, :], r[3:4, :]
    g = gt_ref[0]
    m = g.shape[0]
    gy1, gx1, gy2, gx2 = g[:, 0:1], g[:, 1:2], g[:, 2:3], g[:, 3:4]
    glab = glab_ref[0]

    x_top = jnp.maximum(bx1, gx1)
    y_top = jnp.maximum(by1, gy1)
    x_bot = jnp.minimum(bx2, gx2)
    y_bot = jnp.minimum(by2, gy2)
    inter = jnp.maximum(x_bot - x_top, 0.0) * jnp.maximum(y_bot - y_top, 0.0)
    barea = (by2 - by1) * (bx2 - bx1)
    garea = (gy2 - gy1) * (gx2 - gx1)
    union = barea + garea - inter
    iou = inter / jnp.maximum(union, 1e-7)

    merged = jnp.max(iou, axis=0, keepdims=True)
    iota_g = lax.broadcasted_iota(jnp.int32, (m, n), 0)
    amax = jnp.min(jnp.where(iou == merged, iota_g, m), axis=0, keepdims=True)
    onehot = iota_g == amax

    egy1 = jnp.sum(jnp.where(onehot, gy1, 0.0), axis=0, keepdims=True)
    egx1 = jnp.sum(jnp.where(onehot, gx1, 0.0), axis=0, keepdims=True)
    egy2 = jnp.sum(jnp.where(onehot, gy2, 0.0), axis=0, keepdims=True)
    egx2 = jnp.sum(jnp.where(onehot, gx2, 0.0), axis=0, keepdims=True)
    elab = jnp.sum(jnp.where(onehot, glab, 0), axis=0, keepdims=True)

    mrg_out_ref[0] = merged
    lab_out_ref[0] = elab

    bw = bx2 - bx1
    bh = by2 - by1
    bcx = bx1 + 0.5 * bw
    bcy = by1 + 0.5 * bh
    gw = egx2 - egx1
    gh = egy2 - egy1
    gcx = egx1 + 0.5 * gw
    gcy = egy1 + 0.5 * gh
    bw = jnp.where(bw == 0.0, 0.001, bw)
    bh = jnp.where(bh == 0.0, 0.001, bh)
    dx = jnp.where(gw == 0.0, 0.0, (gcx - bcx) / bw) * 10.0
    dy = jnp.where(gh == 0.0, 0.0, (gcy - bcy) / bh) * 10.0
    dw = jnp.where(gw == 0.0, 0.0, jnp.log(jnp.where(gw == 0.0, 1.0, gw / bw))) * 5.0
    dh = jnp.where(gh == 0.0, 0.0, jnp.log(jnp.where(gh == 0.0, 1.0, gh / bh))) * 5.0
    dlt_out_ref[0] = jnp.concatenate([dy, dx, dh, dw], axis=0)


_GATHER_DNUMS = lax.GatherDimensionNumbers(
    offset_dims=(), collapsed_slice_dims=(0,), start_index_map=(0,))


def _perm(x, idx):
    return lax.gather(x, idx[:, None], _GATHER_DNUMS, (1,),
                      mode=lax.GatherScatterMode.PROMISE_IN_BOUNDS,
                      unique_indices=True, indices_are_sorted=False)


def _sc_select_body(mrg_hbm, rnd_hbm, sel_hbm, mrg_v, rnd_v, mult_v, sel_v):
    wid = lax.axis_index("s") * 2 + lax.axis_index("c")
    nv = _NPAD // 16
    iota16 = lax.iota(jnp.int32, 16)

    def splat_sum(x):
        for sft in (1, 2, 4, 8):
            x = x + _perm(x, iota16 ^ sft)
        return x

    def prefix_incl(x):
        for sft in (1, 2, 4, 8):
            shifted = _perm(x, jnp.maximum(iota16 - sft, 0))
            x = x + jnp.where(iota16 >= sft, shifted, 0)
        return x

    @pl.when(wid < 16)
    def _():
        b = wid >> 1
        kind = wid % 2
        pltpu.sync_copy(mrg_hbm.at[b], mrg_v)
        pltpu.sync_copy(rnd_hbm.at[kind, b], rnd_v)
        k = _POS_K * (1 - kind) + _NEG_K * kind
        kpi = 1 - kind

        def precompute(i, c):
            mrg = mrg_v[pl.ds(i * 16, 16)]
            rnd = rnd_v[pl.ds(i * 16, 16)]
            mp = jnp.where(mrg > _TH, 1, 0)
            mn = jnp.where(mrg < _TH, 1, 0) * jnp.where(mrg >= 0.0, 1, 0)
            msk_i = kpi * mp + (1 - kpi) * mn
            mult_v[pl.ds(i * 16, 16)] = rnd * msk_i
            return c

        lax.fori_loop(0, nv, precompute, jnp.int32(0))

        def count_ge(t):
            def body(i, acc):
                mult = mult_v[pl.ds(i * 16, 16)]
                return acc + jnp.where(mult >= t, 1, 0)
            acc = lax.fori_loop(0, nv, body, jnp.zeros((16,), jnp.int32))
            return splat_sum(acc)

        def bs_body(_, lohi):
            lo, hi = lohi
            mid = (lo + hi) >> 1
            take = count_ge(mid) >= k
            return jnp.where(take, mid, lo), jnp.where(take, hi, mid)

        lo0 = jnp.zeros((16,), jnp.int32)
        hi0 = jnp.zeros((16,), jnp.int32) + 4096
        t, _ = lax.fori_loop(0, 12, bs_body, (lo0, hi0))
        r = k - count_ge(t + 1)

        def tie_body(i, carry):
            mult = mult_v[pl.ds(i * 16, 16)]
            nz_i = jnp.where(mult > 0, 1, 0)
            eq_i = jnp.where(mult == t, 1, 0)
            pref = prefix_incl(eq_i)
            tie_i = eq_i * jnp.where((carry + pref) <= r, 1, 0)
            gt_i = jnp.where(mult > t, 1, 0)
            sel_v[pl.ds(i * 16, 16)] = nz_i * (gt_i + tie_i)
            return carry + splat_sum(eq_i)

        lax.fori_loop(0, nv, tie_body, jnp.zeros((16,), jnp.int32))
        pltpu.sync_copy(sel_v, sel_hbm.at[kind, b])


def _phase2_body(lab_ref, dlt_ref, selp_ref, seln_ref, oh_ref, big_ref):
    lab_g = lab_ref[0]
    dlt = dlt_ref[0]
    sp = selp_ref[0] != 0
    sn = seln_ref[0] != 0
    t = lab_g.shape[0]
    lab = jnp.where(sp, lab_g, -1) + sn.astype(jnp.int32)
    iota_c = lax.broadcasted_iota(jnp.int32, (t, _NUM_LABELS), 1)
    hit = iota_c == lab
    oh_ref[0] = hit.astype(jnp.int32)
    hitp = (hit & sp)[:, None, :]
    dsel = dlt[:, :, None]
    big_ref[0] = jnp.where(hitp, dsel, 0.0)


def kernel(roi_bboxes, gt_boxes, gt_labels):
    b, n = roi_bboxes.shape[0], roi_bboxes.shape[1]
    m = gt_boxes.shape[1]

    key42 = jax.random.key(_RNG_SEED)
    rand_p = jax.random.randint(key42, (b, n), 1, _POS_K * 10, dtype=jnp.int32)
    rand_n = jax.random.randint(key42, (b, n), 1, _NEG_K * 10, dtype=jnp.int32)
    rnd = jnp.stack([rand_p, rand_n])
    rnd = jnp.pad(rnd, ((0, 0), (0, 0), (0, _NPAD - n)))

    roi_t = jnp.transpose(roi_bboxes, (0, 2, 1))
    glab3 = gt_labels.reshape(b, m, 1)

    mrg_row, lab_row, dlt_row = pl.pallas_call(
        _phase1_body,
        grid=(b,),
        in_specs=[
            pl.BlockSpec((1, 4, n), lambda i: (i, 0, 0)),
            pl.BlockSpec((1, m, 4), lambda i: (i, 0, 0)),
            pl.BlockSpec((1, m, 1), lambda i: (i, 0, 0)),
        ],
        out_specs=[
            pl.BlockSpec((1, 1, n), lambda i: (i, 0, 0)),
            pl.BlockSpec((1, 1, n), lambda i: (i, 0, 0)),
            pl.BlockSpec((1, 4, n), lambda i: (i, 0, 0)),
        ],
        out_shape=[
            jax.ShapeDtypeStruct((b, 1, n), jnp.float32),
            jax.ShapeDtypeStruct((b, 1, n), jnp.int32),
            jax.ShapeDtypeStruct((b, 4, n), jnp.float32),
        ],
    )(roi_t, gt_boxes, glab3)

    mrg_pad = jnp.pad(mrg_row.reshape(b, n), ((0, 0), (0, _NPAD - n)),
                      constant_values=-1.0)

    mesh = plsc.VectorSubcoreMesh(core_axis_name="c", subcore_axis_name="s")
    sel = pl.kernel(
        _sc_select_body,
        mesh=mesh,
        out_type=jax.ShapeDtypeStruct((2, b, _NPAD), jnp.int32),
        scratch_types=[
            pltpu.VMEM((_NPAD,), jnp.float32),
            pltpu.VMEM((_NPAD,), jnp.int32),
            pltpu.VMEM((_NPAD,), jnp.int32),
            pltpu.VMEM((_NPAD,), jnp.int32),
        ],
    )(mrg_pad, rnd)

    selp = sel[0, :, :n].reshape(b, n, 1)
    seln = sel[1, :, :n].reshape(b, n, 1)
    lab_col = jnp.transpose(lab_row, (0, 2, 1))
    dlt_col = jnp.transpose(dlt_row, (0, 2, 1))

    tile = 1000
    oh, big_t = pl.pallas_call(
        _phase2_body,
        grid=(b, n // tile),
        in_specs=[
            pl.BlockSpec((1, tile, 1), lambda i, j: (i, j, 0)),
            pl.BlockSpec((1, tile, 4), lambda i, j: (i, j, 0)),
            pl.BlockSpec((1, tile, 1), lambda i, j: (i, j, 0)),
            pl.BlockSpec((1, tile, 1), lambda i, j: (i, j, 0)),
        ],
        out_specs=[
            pl.BlockSpec((1, tile, _NUM_LABELS), lambda i, j: (i, j, 0)),
            pl.BlockSpec((1, tile, 4, _NUM_LABELS), lambda i, j: (i, j, 0, 0)),
        ],
        out_shape=[
            jax.ShapeDtypeStruct((b, n, _NUM_LABELS), jnp.int32),
            jax.ShapeDtypeStruct((b, n, 4, _NUM_LABELS), jnp.float32),
        ],
    )(lab_col, dlt_col, selp, seln)

    big = jnp.swapaxes(big_t, -1, -2)
    return big.reshape(b, n * _NUM_LABELS, 4), oh
